# baseline (speedup 1.0000x reference)
import numpy as np
import jax
import jax.numpy as jnp
from jax import lax
from jax.experimental import pallas as pl
from jax.experimental.pallas import tpu as pltpu
from jax.experimental.pallas import tpu_sc as plsc

_B = 65536
_NUM_LEVELS = 16
_BASE_RES = 16
_LOG2_HASHMAP = 18
_GC = 66

_NC = 2
_NS = 16
_NW = _NC * _NS
_PW = _B // _NW
_GRP = _PW // 16

_P1 = int(np.uint32(2654435761).view(np.int32))
_P2 = int(np.uint32(805459861).view(np.int32))


def _level_consts():
    consts = []
    offset = 0
    max_params = 2 ** _LOG2_HASHMAP
    for l in range(_NUM_LEVELS):
        scale = float(np.exp2(float(l)) * _BASE_RES - 1.0)
        res = int(np.ceil(scale)) + 1
        size_res = int(np.ceil(_BASE_RES * 2.0 ** l))
        params = min(max_params, (size_res + 1) ** 3)
        params = int(np.ceil(params / 8) * 8)
        hashed = res ** 3 > params
        consts.append(dict(scale=np.float32(scale), res=res, hsize=params,
                           base=offset, hashed=hashed))
        offset += params
    return consts

_LEVELS = _level_consts()


def _body(xin, tri, emb, out, xyz_v, tri_v, out_v, idx_v, rows_v, gsem0, gsem1):
    gsems = [gsem0, gsem1]
    wid = lax.axis_index("s") * _NC + lax.axis_index("c")
    base_pt = wid * _PW
    for d in range(3):
        pltpu.sync_copy(xin.at[pl.ds(d * _B + base_pt, _PW)],
                        xyz_v.at[pl.ds(d * _PW, _PW)])
    for r in range(8):
        pltpu.sync_copy(tri.at[pl.ds(r * _B + base_pt, _PW)],
                        tri_v.at[pl.ds(r * _PW, _PW)])

    @pl.loop(0, _GRP)
    def _grp(g):
        s = g * 16
        x = xyz_v[pl.ds(0 * _PW + s, 16)]
        y = xyz_v[pl.ds(1 * _PW + s, 16)]
        z = xyz_v[pl.ds(2 * _PW + s, 16)]
        wA0 = tri_v[pl.ds(0 * _PW + s, 16)]
        wA1 = tri_v[pl.ds(4 * _PW + s, 16)]

        cps = [None, None]

        def fire(l):
            lc = _LEVELS[l]
            buf = l % 2
            scale = jnp.float32(lc["scale"])
            px = x * scale + jnp.float32(0.5)
            py = y * scale + jnp.float32(0.5)
            pz = z * scale + jnp.float32(0.5)
            pgx = px.astype(jnp.int32)
            pgy = py.astype(jnp.int32)
            pgz = pz.astype(jnp.int32)
            if lc["hashed"]:
                mask = jnp.int32(lc["hsize"] - 1)
                h0 = pgx
                h0p = pgx + jnp.int32(1)
                h1 = pgy * jnp.int32(_P1)
                h1p = h1 + jnp.int32(_P1)
                h2 = pgz * jnp.int32(_P2)
                h2p = h2 + jnp.int32(_P2)
            else:
                res = lc["res"]
                h0 = pgx
                h0p = pgx + jnp.int32(1)
                h1 = pgy * jnp.int32(res)
                h1p = h1 + jnp.int32(res)
                h2 = pgz * jnp.int32(res * res)
                h2p = h2 + jnp.int32(res * res)
            for corner in range(8):
                a = h0p if (corner & 1) else h0
                b = h1p if (corner & 2) else h1
                c = h2p if (corner & 4) else h2
                if lc["hashed"]:
                    row = ((a ^ b ^ c) & mask) + jnp.int32(lc["base"])
                else:
                    sidx = a + b + c
                    sidx = jnp.where(sidx >= jnp.int32(lc["hsize"]),
                                     sidx - jnp.int32(lc["hsize"]), sidx)
                    row = sidx + jnp.int32(lc["base"])
                idx_v[pl.ds(buf * 128 + corner * 16, 16)] = row
            cps[buf] = pltpu.async_copy(
                emb.at[idx_v.at[pl.ds(buf * 128, 128)]],
                rows_v.at[pl.ds(buf * 128, 128)], gsems[buf])
            return px

        def fake_interp(l, px):
            acc0 = px * wA0
            acc1 = px * wA1
            out_v[pl.ds((2 * l) * _PW + s, 16)] = acc0
            out_v[pl.ds((2 * l + 1) * _PW + s, 16)] = acc1

        pxs = [None] * 16
        pxs[0] = fire(0)
        for l in range(1, _NUM_LEVELS):
            pxs[l] = fire(l)
            cps[(l - 1) % 2].wait()
            fake_interp(l - 1, pxs[l - 1])
        cps[15 % 2].wait()
        fake_interp(15, pxs[15])

    for j in range(2 * _NUM_LEVELS):
        pltpu.sync_copy(out_v.at[pl.ds(j * _PW, _PW)],
                        out.at[pl.ds(j * _B + base_pt, _PW)])


@jax.jit
def _encode(xin, tri, emb):
    mesh = plsc.VectorSubcoreMesh(core_axis_name="c", subcore_axis_name="s")
    f = pl.kernel(
        _body,
        out_type=jax.ShapeDtypeStruct((2 * _NUM_LEVELS * _B,), jnp.float32),
        mesh=mesh,
        compiler_params=pltpu.CompilerParams(use_tc_tiling_on_sc=False),
        scratch_types=[
            pltpu.VMEM((3 * _PW,), jnp.float32),
            pltpu.VMEM((8 * _PW,), jnp.float32),
            pltpu.VMEM((2 * _NUM_LEVELS * _PW,), jnp.float32),
            pltpu.VMEM((2 * 128,), jnp.int32),
            pltpu.VMEM((2 * 128, _GC), jnp.float32),
            pltpu.SemaphoreType.DMA,
            pltpu.SemaphoreType.DMA,
        ],
    )
    return f(xin, tri, emb)


def kernel(inputs, temporal_row_index, embeddings):
    xin = inputs.T.reshape(-1)
    tri = temporal_row_index.T.reshape(-1)
    out_t = _encode(xin, tri, embeddings)
    return out_t.reshape(2 * _NUM_LEVELS, _B).T


# speedup vs baseline: 1.1927x; 1.1927x over previous
import numpy as np
import jax
import jax.numpy as jnp
from jax import lax
from jax.experimental import pallas as pl
from jax.experimental.pallas import tpu as pltpu
from jax.experimental.pallas import tpu_sc as plsc

_B = 65536
_NUM_LEVELS = 16
_BASE_RES = 16
_LOG2_HASHMAP = 18
_GC = 66

_NC = 2
_NS = 16
_NW = _NC * _NS
_PW = _B // _NW
_GRP = _PW // 16
_NIDX = _NUM_LEVELS * 8 * 4 * 16

_P1 = int(np.uint32(2654435761).view(np.int32))
_P2 = int(np.uint32(805459861).view(np.int32))

_L0N = ((4920 * _GC + 127) // 128) * 128


def _level_consts():
    consts = []
    offset = 0
    max_params = 2 ** _LOG2_HASHMAP
    for l in range(_NUM_LEVELS):
        scale = float(np.exp2(float(l)) * _BASE_RES - 1.0)
        res = int(np.ceil(scale)) + 1
        size_res = int(np.ceil(_BASE_RES * 2.0 ** l))
        params = min(max_params, (size_res + 1) ** 3)
        params = int(np.ceil(params / 8) * 8)
        hashed = res ** 3 > params
        consts.append(dict(scale=np.float32(scale), res=res, hsize=params,
                           base=offset, hashed=hashed))
        offset += params
    return consts

_LEVELS = _level_consts()


def _body(xin, tri, emb, out, xyz_v, tri_v, out_v, w_v, idx_v, gat_v, sp0_v,
          gsem0, gsem1, gsem2, gsem3, gsem4):
    gsems = [gsem0, gsem1, gsem2, gsem3]
    sid = lax.axis_index("s")
    wid = sid * _NC + lax.axis_index("c")
    base_pt = wid * _PW

    @pl.when(sid == 0)
    def _stage():
        pltpu.sync_copy(emb.at[pl.ds(0, _L0N)], sp0_v)

    plsc.subcore_barrier()
    for d in range(3):
        pltpu.sync_copy(xin.at[pl.ds(d * _B + base_pt, _PW)],
                        xyz_v.at[pl.ds(d * _PW, _PW)])
    for r in range(8):
        pltpu.sync_copy(tri.at[pl.ds(r * _B + base_pt, _PW)],
                        tri_v.at[pl.ds(r * _PW, _PW)])

    @pl.loop(0, _GRP)
    def _grp(g):
        s = g * 16
        x = xyz_v[pl.ds(0 * _PW + s, 16)]
        y = xyz_v[pl.ds(1 * _PW + s, 16)]
        z = xyz_v[pl.ds(2 * _PW + s, 16)]
        cA0 = tri_v[pl.ds(1 * _PW + s, 16)].astype(jnp.int32)
        cB0 = tri_v[pl.ds(3 * _PW + s, 16)].astype(jnp.int32)
        cA1 = tri_v[pl.ds(5 * _PW + s, 16)].astype(jnp.int32)
        cB1 = tri_v[pl.ds(7 * _PW + s, 16)].astype(jnp.int32)

        cps = [None] * 4
        blk = _NIDX // 4
        for l, lc in enumerate(_LEVELS):
            scale = jnp.float32(lc["scale"])
            px = x * scale + jnp.float32(0.5)
            py = y * scale + jnp.float32(0.5)
            pz = z * scale + jnp.float32(0.5)
            pgx = px.astype(jnp.int32)
            pgy = py.astype(jnp.int32)
            pgz = pz.astype(jnp.int32)
            fx = px - pgx.astype(jnp.float32)
            fy = py - pgy.astype(jnp.float32)
            fz = pz - pgz.astype(jnp.float32)

            ox = jnp.float32(1.0) - fx
            oy = jnp.float32(1.0) - fy
            oz = jnp.float32(1.0) - fz
            wxy = [ox * oy, fx * oy, ox * fy, fx * fy]

            if lc["hashed"]:
                mask = jnp.int32(lc["hsize"] - 1)
                h0 = pgx
                h0p = pgx + jnp.int32(1)
                h1 = pgy * jnp.int32(_P1)
                h1p = h1 + jnp.int32(_P1)
                h2 = pgz * jnp.int32(_P2)
                h2p = h2 + jnp.int32(_P2)
            else:
                res = lc["res"]
                h0 = pgx
                h0p = pgx + jnp.int32(1)
                h1 = pgy * jnp.int32(res)
                h1p = h1 + jnp.int32(res)
                h2 = pgz * jnp.int32(res * res)
                h2p = h2 + jnp.int32(res * res)

            for corner in range(8):
                a = h0p if (corner & 1) else h0
                b = h1p if (corner & 2) else h1
                c = h2p if (corner & 4) else h2
                if lc["hashed"]:
                    row = ((a ^ b ^ c) & mask) + jnp.int32(lc["base"])
                else:
                    sidx = a + b + c
                    sidx = jnp.where(sidx >= jnp.int32(lc["hsize"]),
                                     sidx - jnp.int32(lc["hsize"]), sidx)
                    row = sidx + jnp.int32(lc["base"])
                e = row * jnp.int32(_GC)
                ibase = ((l * 8 + corner) * 4) * 16
                idx_v[pl.ds(ibase, 16)] = e + cA0
                idx_v[pl.ds(ibase + 16, 16)] = e + cB0
                idx_v[pl.ds(ibase + 32, 16)] = e + cA1
                idx_v[pl.ds(ibase + 48, 16)] = e + cB1
                w = wxy[corner & 3] * (fz if (corner & 4) else oz)
                w_v[pl.ds((l * 8 + corner) * 16, 16)] = w
            if l == 0:
                cps.append(pltpu.async_copy(
                    sp0_v.at[idx_v.at[pl.ds(0, 512)]],
                    gat_v.at[pl.ds(0, 512)], gsem4))
            if l % 4 == 3:
                k = l // 4
                start = k * blk if k else 512
                cps[k] = pltpu.async_copy(
                    emb.at[idx_v.at[pl.ds(start, (k + 1) * blk - start)]],
                    gat_v.at[pl.ds(start, (k + 1) * blk - start)], gsems[k])

        wA0 = tri_v[pl.ds(0 * _PW + s, 16)]
        wB0 = tri_v[pl.ds(2 * _PW + s, 16)]
        wA1 = tri_v[pl.ds(4 * _PW + s, 16)]
        wB1 = tri_v[pl.ds(6 * _PW + s, 16)]

        for l in range(_NUM_LEVELS):
            if l == 0:
                cps[4].wait()
            elif l == 1:
                cps[0].wait()
            elif l % 4 == 0:
                cps[l // 4].wait()
            acc0 = None
            acc1 = None
            for corner in range(8):
                ibase = ((l * 8 + corner) * 4) * 16
                w = w_v[pl.ds((l * 8 + corner) * 16, 16)]
                gA0 = gat_v[pl.ds(ibase, 16)]
                gB0 = gat_v[pl.ds(ibase + 16, 16)]
                gA1 = gat_v[pl.ds(ibase + 32, 16)]
                gB1 = gat_v[pl.ds(ibase + 48, 16)]
                t0 = w * (wA0 * gA0 + wB0 * gB0)
                t1 = w * (wA1 * gA1 + wB1 * gB1)
                acc0 = t0 if acc0 is None else acc0 + t0
                acc1 = t1 if acc1 is None else acc1 + t1
            out_v[pl.ds((2 * l) * _PW + s, 16)] = acc0
            out_v[pl.ds((2 * l + 1) * _PW + s, 16)] = acc1

    for j in range(2 * _NUM_LEVELS):
        pltpu.sync_copy(out_v.at[pl.ds(j * _PW, _PW)],
                        out.at[pl.ds(j * _B + base_pt, _PW)])


@jax.jit
def _encode(xin, tri, emb):
    mesh = plsc.VectorSubcoreMesh(core_axis_name="c", subcore_axis_name="s")
    f = pl.kernel(
        _body,
        out_type=jax.ShapeDtypeStruct((2 * _NUM_LEVELS * _B,), jnp.float32),
        mesh=mesh,
        scratch_types=[
            pltpu.VMEM((3 * _PW,), jnp.float32),
            pltpu.VMEM((8 * _PW,), jnp.float32),
            pltpu.VMEM((2 * _NUM_LEVELS * _PW,), jnp.float32),
            pltpu.VMEM((_NUM_LEVELS * 8 * 16,), jnp.float32),
            pltpu.VMEM((_NIDX,), jnp.int32),
            pltpu.VMEM((_NIDX,), jnp.float32),
            pltpu.VMEM_SHARED((_L0N,), jnp.float32),
            pltpu.SemaphoreType.DMA,
            pltpu.SemaphoreType.DMA,
            pltpu.SemaphoreType.DMA,
            pltpu.SemaphoreType.DMA,
            pltpu.SemaphoreType.DMA,
        ],
    )
    return f(xin, tri, emb)


def kernel(inputs, temporal_row_index, embeddings):
    xin = inputs.T.reshape(-1)
    tri = temporal_row_index.T.reshape(-1)
    out_t = _encode(xin, tri, embeddings.reshape(-1))
    return out_t.reshape(2 * _NUM_LEVELS, _B).T
